# baseline (speedup 1.0000x reference)
import jax
import jax.numpy as jnp
from jax.experimental import pallas as pl
from jax.experimental.pallas import tpu as pltpu

_CHUNKS = 4


def _dma_body(x_hbm, o_hbm, sem):
    n = x_hbm.shape[0]
    c = n // _CHUNKS
    for i in range(_CHUNKS):
        pltpu.make_async_copy(
            x_hbm.at[pl.ds(i * c, c)], o_hbm.at[pl.ds(i * c, c)], sem
        ).start()
    for i in range(_CHUNKS):
        pltpu.make_async_copy(
            x_hbm.at[pl.ds(i * c, c)], o_hbm.at[pl.ds(i * c, c)], sem
        ).wait()


def kernel(x, question_mask, table_mask, column_mask):
    n, d = x.shape
    return pl.pallas_call(
        _dma_body,
        in_specs=[pl.BlockSpec(memory_space=pl.ANY)],
        out_specs=pl.BlockSpec(memory_space=pl.ANY),
        out_shape=jax.ShapeDtypeStruct((n, d), x.dtype),
        scratch_shapes=[pltpu.SemaphoreType.DMA],
    )(x)


# speedup vs baseline: 49.0853x; 49.0853x over previous
import jax
import jax.numpy as jnp
from jax.experimental import pallas as pl


def _copy_body(x_ref, o_ref):
    o_ref[...] = x_ref[...]


def kernel(x, question_mask, table_mask, column_mask):
    n, d = x.shape
    blk = 4096
    return pl.pallas_call(
        _copy_body,
        grid=(n // blk,),
        in_specs=[pl.BlockSpec((blk, d), lambda i: (i, 0))],
        out_specs=pl.BlockSpec((blk, d), lambda i: (i, 0)),
        out_shape=jax.ShapeDtypeStruct((n, d), x.dtype),
    )(x)
